# Initial kernel scaffold; baseline (speedup 1.0000x reference)
import jax
import jax.numpy as jnp
from jax.experimental import pallas as pl


def kernel(x, edge_index, W1, b1, W2, b2):
    raise NotImplementedError("write your pallas kernel here")


# speedup vs baseline: 16.1492x; 16.1492x over previous
import functools

import jax
import jax.numpy as jnp
from jax import lax
from jax.experimental import pallas as pl
from jax.experimental.pallas import tpu as pltpu
from jax.experimental.pallas import tpu_sc as plsc

N = 10000
E = 160000
NC = 2
NS = 16
LANES = 128
HALF = 128
ROWS = E // LANES
RPAD = 1280
NPAD = 10240
BM = 1000


def _mesh():
    return plsc.VectorSubcoreMesh(
        core_axis_name="c", subcore_axis_name="s", num_cores=NC, num_subcores=NS
    )


def _make_degree_kernel():
    load = RPAD // (NC * NS)

    @functools.partial(
        pl.kernel,
        out_type=jax.ShapeDtypeStruct((NC * NPAD,), jnp.float32),
        mesh=_mesh(),
        scratch_types=[
            pltpu.VMEM((load, LANES), jnp.int32),
            pltpu.VMEM((LANES,), jnp.float32),
            pltpu.VMEM((NPAD // NS,), jnp.float32),
            pltpu.VMEM_SHARED((NPAD,), jnp.float32),
        ],
    )
    def deg_kernel(dstp_hbm, zo_hbm, out_hbm, idxd_v, ones_v, stage_v, acc_sh):
        cid = lax.axis_index("c")
        sid = lax.axis_index("s")
        t = cid * NS + sid
        seg = NPAD // NS

        pltpu.sync_copy(zo_hbm.at[pl.ds(0, seg)], stage_v)
        pltpu.sync_copy(zo_hbm.at[pl.ds(seg, LANES)], ones_v)
        pltpu.sync_copy(stage_v, acc_sh.at[pl.ds(sid * seg, seg)])
        base = load * t
        count = jnp.minimum(load, jnp.maximum(ROWS - base, 0))
        pltpu.sync_copy(dstp_hbm.at[pl.ds(base, load)], idxd_v)
        plsc.subcore_barrier()

        def body(j, carry):
            pltpu.sync_copy(ones_v, acc_sh.at[idxd_v.at[j]], add=True)
            return carry

        lax.fori_loop(0, count, body, 0)
        plsc.subcore_barrier()
        pltpu.sync_copy(acc_sh.at[pl.ds(sid * seg, seg)], stage_v)
        pltpu.sync_copy(stage_v, out_hbm.at[pl.ds(cid * NPAD + sid * seg, seg)])

    return deg_kernel


def _make_prop_kernel():
    load = RPAD // NS

    @functools.partial(
        pl.kernel,
        out_type=jax.ShapeDtypeStruct((NC * NPAD, HALF), jnp.float32),
        mesh=_mesh(),
        scratch_types=[
            pltpu.VMEM((load, LANES), jnp.int32),
            pltpu.VMEM((load, LANES), jnp.int32),
            pltpu.VMEM((LANES, HALF), jnp.float32),
            pltpu.VMEM_SHARED((NPAD, HALF), jnp.float32),
            pltpu.SemaphoreType.DMA,
        ],
    )
    def prop_kernel(
        xc2_hbm, srcp_hbm, dstp_hbm, z2_hbm, out_hbm, idxs_v, idxd_v, rows_v, acc_sh, sem
    ):
        cid = lax.axis_index("c")
        sid = lax.axis_index("s")
        seg = NPAD // NS

        pltpu.sync_copy(z2_hbm, rows_v)
        for j in range(seg // LANES):
            pltpu.sync_copy(rows_v, acc_sh.at[pl.ds(sid * seg + j * LANES, LANES)])

        base = load * sid
        count = jnp.minimum(load, jnp.maximum(ROWS - base, 0))
        pltpu.sync_copy(srcp_hbm.at[pl.ds(cid * RPAD + base, load)], idxs_v)
        pltpu.sync_copy(dstp_hbm.at[pl.ds(base, load)], idxd_v)
        plsc.subcore_barrier()

        def body(j, carry):
            pltpu.async_copy(xc2_hbm.at[idxs_v.at[j]], rows_v, sem).wait()
            pltpu.sync_copy(rows_v, acc_sh.at[idxd_v.at[j]], add=True)
            return carry

        lax.fori_loop(0, count, body, 0)
        plsc.subcore_barrier()
        for j in range(seg // LANES):
            pltpu.sync_copy(acc_sh.at[pl.ds(sid * seg + j * LANES, LANES)], rows_v)
            pltpu.sync_copy(
                rows_v, out_hbm.at[pl.ds(cid * NPAD + sid * seg + j * LANES, LANES)]
            )

    return prop_kernel


def _dv_block(degt_blk):
    return lax.rsqrt(degt_blk[:, 0:1] + degt_blk[:, 1:2] + 1.0)


def _tc_prep(degt, x):
    def body(degt_ref, x_ref, out_ref):
        dv = _dv_block(degt_ref[...])
        xb = x_ref[...]
        out_ref[0] = xb[:, :HALF] * dv
        out_ref[1] = xb[:, HALF:] * dv

    return pl.pallas_call(
        body,
        grid=(N // BM,),
        in_specs=[
            pl.BlockSpec((BM, 2), lambda i: (i, 0)),
            pl.BlockSpec((BM, 2 * HALF), lambda i: (i, 0)),
        ],
        out_specs=pl.BlockSpec((2, BM, HALF), lambda i: (0, i, 0)),
        out_shape=jax.ShapeDtypeStruct((2, N, HALF), jnp.float32),
    )(degt, x)


def _tc_main(degt, S1, XC, W1, b1, W2):
    def body(degt_ref, s1_ref, xc_ref, w1_ref, b1_ref, w2_ref, out_ref):
        dv = _dv_block(degt_ref[...])
        p = jnp.concatenate(
            [(s1_ref[0] + xc_ref[0]) * dv, (s1_ref[1] + xc_ref[1]) * dv], axis=1
        )
        h = jnp.dot(p, w1_ref[...], preferred_element_type=jnp.float32)
        h = jnp.maximum(h + b1_ref[...], 0.0)
        y = jnp.dot(h, w2_ref[...], preferred_element_type=jnp.float32)
        out_ref[0] = y[:, :HALF] * dv
        out_ref[1] = y[:, HALF:] * dv

    return pl.pallas_call(
        body,
        grid=(N // BM,),
        in_specs=[
            pl.BlockSpec((BM, 2), lambda i: (i, 0)),
            pl.BlockSpec((2, BM, HALF), lambda i: (0, i, 0)),
            pl.BlockSpec((2, BM, HALF), lambda i: (0, i, 0)),
            pl.BlockSpec(W1.shape, lambda i: (0, 0)),
            pl.BlockSpec((1, W1.shape[1]), lambda i: (0, 0)),
            pl.BlockSpec(W2.shape, lambda i: (0, 0)),
        ],
        out_specs=pl.BlockSpec((2, BM, HALF), lambda i: (0, i, 0)),
        out_shape=jax.ShapeDtypeStruct((2, N, HALF), jnp.float32),
    )(degt, S1, XC, W1, b1, W2)


def _tc_final(degt, S2, Y2, b2):
    def body(degt_ref, s2_ref, y2_ref, b2_ref, out_ref):
        dv = _dv_block(degt_ref[...])
        out_ref[...] = (
            jnp.concatenate(
                [(s2_ref[0] + y2_ref[0]) * dv, (s2_ref[1] + y2_ref[1]) * dv], axis=1
            )
            + b2_ref[...]
        )

    return pl.pallas_call(
        body,
        grid=(N // BM,),
        in_specs=[
            pl.BlockSpec((BM, 2), lambda i: (i, 0)),
            pl.BlockSpec((2, BM, HALF), lambda i: (0, i, 0)),
            pl.BlockSpec((2, BM, HALF), lambda i: (0, i, 0)),
            pl.BlockSpec((1, 2 * HALF), lambda i: (0, 0)),
        ],
        out_specs=pl.BlockSpec((BM, 2 * HALF), lambda i: (i, 0)),
        out_shape=jax.ShapeDtypeStruct((N, 2 * HALF), jnp.float32),
    )(degt, S2, Y2, b2)


def kernel(x, edge_index, W1, b1, W2, b2):
    src = edge_index[0].astype(jnp.int32)
    dst = edge_index[1].astype(jnp.int32)
    pad = ((0, 0), (0, RPAD - ROWS), (0, 0))
    srcp = jnp.pad(
        jnp.stack([src, src + N]).reshape(NC, ROWS, LANES), pad
    ).reshape(NC * RPAD, LANES)
    dstp = jnp.pad(dst.reshape(1, ROWS, LANES), pad).reshape(RPAD, LANES)
    zo = jnp.concatenate(
        [jnp.zeros((NPAD // NS,), jnp.float32), jnp.ones((LANES,), jnp.float32)]
    )
    z2 = jnp.zeros((LANES, HALF), jnp.float32)

    degp = _make_degree_kernel()(dstp, zo)
    degt = degp.reshape(NC, NPAD)[:, :N].T

    XC = _tc_prep(degt, x)
    prop = _make_prop_kernel()
    S1 = prop(XC.reshape(NC * N, HALF), srcp, dstp, z2)
    S1 = S1.reshape(NC, NPAD, HALF)[:, :N, :]
    Y2 = _tc_main(degt, S1, XC, W1, b1.reshape(1, -1), W2)
    S2 = prop(Y2.reshape(NC * N, HALF), srcp, dstp, z2)
    S2 = S2.reshape(NC, NPAD, HALF)[:, :N, :]
    return _tc_final(degt, S2, Y2, b2.reshape(1, -1))
